# baseline (speedup 1.0000x reference)
import functools

import jax
import jax.numpy as jnp
from jax import lax
from jax.experimental import pallas as pl
from jax.experimental.pallas import tpu as pltpu
from jax.experimental.pallas import tpu_sc as plsc

B, T, D = 16, 4096, 1024
NC, NS, L = 2, 16, 16
NW = NC * NS
CH = 64
NCH = T // CH
SLOTS = NCH // NW
ND = D // L


def _sc_partials_kernel(payload_hbm, lens_hbm, partials_hbm, lens_v, buf_v,
                        acc_v):
    w = lax.axis_index("s") * NC + lax.axis_index("c")
    pltpu.sync_copy(lens_hbm, lens_v)
    lens_vec = lens_v[...]

    for b in range(B):
        len_b = lens_vec[b]

        def zbody(d, _):
            acc_v[pl.ds(d * L, L)] = jnp.zeros((L,), jnp.float32)
            return 0
        lax.fori_loop(0, ND, zbody, 0)

        c0 = lax.rem(w + 2 * (NW - b), NW)
        for kk in range(SLOTS):
            start = (c0 + kk * NW) * CH
            nv = jnp.clip(len_b - start, 0, CH)

            @pl.when(nv > 0)
            def _():
                pltpu.sync_copy(payload_hbm.at[b, pl.ds(start, CH), :], buf_v)
                n8i = nv // 8

                def dbody(d, _):
                    ds_ = pl.ds(d * L, L)
                    reg = acc_v[ds_]

                    def r8(r, acc):
                        for u in range(8):
                            acc = acc + buf_v[r + u, ds_]
                        return acc
                    reg = lax.fori_loop(0, n8i, lambda i, a: r8(i * 8, a),
                                        reg)

                    def r1(r, acc):
                        return acc + buf_v[r, ds_]
                    reg = lax.fori_loop(n8i * 8, nv, r1, reg)
                    acc_v[ds_] = reg
                    return 0
                lax.fori_loop(0, ND, dbody, 0)

        pltpu.sync_copy(acc_v, partials_hbm.at[w, b, :])


def _tc_combine_kernel(partials_ref, lens_ref, out_ref):
    s = jnp.sum(partials_ref[...], axis=0)
    out_ref[...] = s / lens_ref[...]


@jax.jit
def kernel(payload, seq_lens):
    lens_i32 = seq_lens.astype(jnp.int32)

    mesh = plsc.VectorSubcoreMesh(core_axis_name="c", subcore_axis_name="s",
                                  num_cores=NC, num_subcores=NS)
    partials = pl.kernel(
        _sc_partials_kernel,
        out_type=jax.ShapeDtypeStruct((NW, B, D), jnp.float32),
        mesh=mesh,
        scratch_types=[
            pltpu.VMEM((L,), jnp.int32),
            pltpu.VMEM((CH, D), jnp.float32),
            pltpu.VMEM((D,), jnp.float32),
        ],
    )(payload, lens_i32)

    lens_f = lens_i32.astype(jnp.float32).reshape(B, 1)
    out = pl.pallas_call(
        _tc_combine_kernel,
        out_shape=jax.ShapeDtypeStruct((B, D), jnp.float32),
    )(partials, lens_f)
    return out
